# baseline (speedup 1.0000x reference)
import jax, jax.numpy as jnp
from jax.experimental import pallas as pl
from jax.experimental.pallas import tpu as pltpu

def _body(x_ref, out_ref):
    i = pl.program_id(0)
    xb = x_ref[...]
    out_ref[0, 0] = jnp.max(xb)

def kernel(inputs, targets, alpha):
    x2 = inputs.reshape(16000, 1024)
    out = pl.pallas_call(
        _body,
        grid=(8,),
        in_specs=[pl.BlockSpec((2000, 1024), lambda i: (i, 0))],
        out_specs=pl.BlockSpec(memory_space=pltpu.SMEM),
        out_shape=jax.ShapeDtypeStruct((1, 1), jnp.float32),
    )(x2)
    return out[0, 0]
